# baseline (speedup 1.0000x reference)
import functools

import jax
import jax.numpy as jnp
from jax import lax
from jax.experimental import pallas as pl
from jax.experimental.pallas import tpu as pltpu
from jax.experimental.pallas import tpu_sc as plsc

_N = 200000
_C = 80
_CHUNKS = 1563
_NW = 32


_FULL = _CHUNKS - 1
_TAILW = _N - _FULL * 128


@functools.partial(
    pl.kernel,
    out_type=jax.ShapeDtypeStruct((_FULL * 128 + 128,), jnp.float32),
    mesh=plsc.VectorSubcoreMesh(core_axis_name="c", subcore_axis_name="s"),
    scratch_types=[
        pltpu.VMEM((_C, 128), jnp.float32),
        pltpu.VMEM((_C, _TAILW), jnp.float32),
        pltpu.VMEM((128,), jnp.int32),
        pltpu.VMEM((128,), jnp.float32),
    ],
    compiler_params=pltpu.CompilerParams(use_tc_tiling_on_sc=True, needs_layout_passes=False),
)
def _gs_gather(gt_hbm, tail_hbm, lab_hbm, gs_hbm, chunk_v, tail_v, lab_v, gs_v):
    w = lax.axis_index("s") * 2 + lax.axis_index("c")
    nt = jnp.where(w < _FULL - (_FULL // _NW) * _NW, _FULL // _NW + 1,
                   _FULL // _NW)

    def body(t, carry):
        j = w + t * _NW
        col0 = pl.multiple_of(j * 128, 128)
        pltpu.sync_copy(lab_hbm.at[pl.ds(col0, 128)], lab_v)
        pltpu.sync_copy(gt_hbm.at[:, pl.ds(col0, 128)], chunk_v)
        for k in range(8):
            lab16 = lab_v[pl.ds(k * 16, 16)]
            cols = lax.iota(jnp.int32, 16) + (k * 16)
            gs_v[pl.ds(k * 16, 16)] = plsc.load_gather(chunk_v, [lab16, cols])
        pltpu.sync_copy(gs_v, gs_hbm.at[pl.ds(col0, 128)])
        return carry

    lax.fori_loop(0, nt, body, 0)

    @pl.when(w == _FULL % _NW)
    def _tail():
        base = _FULL * 128
        pltpu.sync_copy(lab_hbm.at[pl.ds(base, 128)], lab_v)
        pltpu.sync_copy(tail_hbm, tail_v)
        for k in range(_TAILW // 16):
            lab16 = lab_v[pl.ds(k * 16, 16)]
            cols = lax.iota(jnp.int32, 16) + (k * 16)
            gs_v[pl.ds(k * 16, 16)] = plsc.load_gather(tail_v, [lab16, cols])
        pltpu.sync_copy(gs_v, gs_hbm.at[pl.ds(base, 128)])


def _qfl_block_t(pred_ref, gs_ref, label_ref, out_ref):
    x = pred_ref[...]
    gs = gs_ref[...]
    lab = label_ref[...]
    C, B = x.shape

    rows = lax.broadcasted_iota(jnp.int32, (C, B), 0)
    onehot = rows == lab

    th = jnp.tanh(0.5 * x)
    s = 0.5 * th + 0.5
    s_abs = 0.5 * jnp.abs(th) + 0.5
    sp = -jnp.log(s_abs)
    base = jnp.maximum(x, 0.0) + sp

    a = jnp.where(onehot, gs, 0.0)
    t = jnp.where(onehot, gs - s, s)
    out_ref[...] = (base - x * a) * (t * t)


@jax.jit
def kernel(pred_logits, gt_label, gt_score):
    N, C = pred_logits.shape
    BN = 20480
    grid = (pl.cdiv(N, BN),)
    pt = pred_logits.T
    gt = gt_score.T
    lab = gt_label.astype(jnp.int32)
    lab_p = jnp.pad(lab, (0, _FULL * 128 + 128 - N))
    tail = gt[:, _FULL * 128:]
    gs = _gs_gather(gt, tail, lab_p)
    out_t = pl.pallas_call(
        _qfl_block_t,
        grid=grid,
        in_specs=[
            pl.BlockSpec((C, BN), lambda i: (0, i)),
            pl.BlockSpec((BN,), lambda i: (i,)),
            pl.BlockSpec((BN,), lambda i: (i,)),
        ],
        out_specs=pl.BlockSpec((C, BN), lambda i: (0, i)),
        out_shape=jax.ShapeDtypeStruct((C, N), jnp.float32),
    )(pt, gs, lab)
    return out_t.T


# speedup vs baseline: 1.4224x; 1.4224x over previous
import functools

import jax
import jax.numpy as jnp
from jax import lax
from jax.experimental import pallas as pl
from jax.experimental.pallas import tpu as pltpu
from jax.experimental.pallas import tpu_sc as plsc

_N = 200000
_C = 80
_CW = 1024
_FULL = _N // _CW
_TAILW = _N - _FULL * _CW
_NP = _FULL * _CW + _CW
_NW = 32


@functools.partial(
    pl.kernel,
    out_type=jax.ShapeDtypeStruct((_NP,), jnp.float32),
    mesh=plsc.VectorSubcoreMesh(core_axis_name="c", subcore_axis_name="s"),
    scratch_types=[
        pltpu.VMEM((_C, _CW), jnp.float32),
        pltpu.VMEM((_C, _TAILW), jnp.float32),
        pltpu.VMEM((_CW,), jnp.int32),
        pltpu.VMEM((_CW,), jnp.float32),
    ],
    compiler_params=pltpu.CompilerParams(
        use_tc_tiling_on_sc=True, needs_layout_passes=False),
)
def _gs_gather(gt_hbm, tail_hbm, lab_hbm, gs_hbm, chunk_v, tail_v, lab_v, gs_v):
    w = lax.axis_index("s") * 2 + lax.axis_index("c")
    nt = jnp.where(w < _FULL - (_FULL // _NW) * _NW, _FULL // _NW + 1,
                   _FULL // _NW)

    def body(t, carry):
        j = w + t * _NW
        col0 = pl.multiple_of(j * _CW, _CW)
        pltpu.sync_copy(lab_hbm.at[pl.ds(col0, _CW)], lab_v)
        pltpu.sync_copy(gt_hbm.at[:, pl.ds(col0, _CW)], chunk_v)
        for k in range(_CW // 16):
            lab16 = lab_v[pl.ds(k * 16, 16)]
            cols = lax.iota(jnp.int32, 16) + (k * 16)
            gs_v[pl.ds(k * 16, 16)] = plsc.load_gather(chunk_v, [lab16, cols])
        pltpu.sync_copy(gs_v, gs_hbm.at[pl.ds(col0, _CW)])
        return carry

    lax.fori_loop(0, nt, body, 0)

    @pl.when(w == _FULL % _NW)
    def _tail():
        base = _FULL * _CW
        pltpu.sync_copy(lab_hbm.at[pl.ds(base, _CW)], lab_v)
        pltpu.sync_copy(tail_hbm, tail_v)
        for k in range(_TAILW // 16):
            lab16 = lab_v[pl.ds(k * 16, 16)]
            cols = lax.iota(jnp.int32, 16) + (k * 16)
            gs_v[pl.ds(k * 16, 16)] = plsc.load_gather(tail_v, [lab16, cols])
        pltpu.sync_copy(gs_v.at[pl.ds(0, _TAILW)], gs_hbm.at[pl.ds(base, _TAILW)])


def _qfl_block_t(pred_ref, gs_ref, label_ref, out_ref):
    x = pred_ref[...]
    gs = gs_ref[...]
    lab = label_ref[...]
    C, B = x.shape

    rows = lax.broadcasted_iota(jnp.int32, (C, B), 0)
    onehot = rows == lab

    th = jnp.tanh(0.5 * x)
    s = 0.5 * th + 0.5
    s_abs = 0.5 * jnp.abs(th) + 0.5
    sp = -jnp.log(s_abs)
    base = jnp.maximum(x, 0.0) + sp

    a = jnp.where(onehot, gs, 0.0)
    t = jnp.where(onehot, gs - s, s)
    out_ref[...] = (base - x * a) * (t * t)


@jax.jit
def kernel(pred_logits, gt_label, gt_score):
    N, C = pred_logits.shape
    BN = 20480
    grid = (pl.cdiv(N, BN),)
    pt = pred_logits.T
    gt = gt_score.T
    lab = gt_label.astype(jnp.int32)
    lab_p = jnp.pad(lab, (0, _NP - N))
    tail = gt[:, _FULL * _CW:]
    gs = _gs_gather(gt, tail, lab_p)
    out_t = pl.pallas_call(
        _qfl_block_t,
        grid=grid,
        in_specs=[
            pl.BlockSpec((C, BN), lambda i: (0, i)),
            pl.BlockSpec((BN,), lambda i: (i,)),
            pl.BlockSpec((BN,), lambda i: (i,)),
        ],
        out_specs=pl.BlockSpec((C, BN), lambda i: (0, i)),
        out_shape=jax.ShapeDtypeStruct((C, N), jnp.float32),
    )(pt, gs, lab)
    return out_t.T


# speedup vs baseline: 2.2499x; 1.5818x over previous
import functools

import jax
import jax.numpy as jnp
from jax import lax
from jax.experimental import pallas as pl
from jax.experimental.pallas import tpu as pltpu


def _qfl_block_t(pred_ref, gts_ref, label_ref, out_ref):
    x = pred_ref[...]
    g = gts_ref[...]
    lab = label_ref[...]
    C, B = x.shape

    rows = lax.broadcasted_iota(jnp.int32, (C, B), 0)
    onehot = rows == lab

    th = jnp.tanh(0.5 * x)
    s = 0.5 * th + 0.5
    s_abs = 0.5 * jnp.abs(th) + 0.5
    sp = -jnp.log(s_abs)
    base = jnp.maximum(x, 0.0) + sp

    a = jnp.where(onehot, g, 0.0)
    t = jnp.where(onehot, g - s, s)
    out_ref[...] = (base - x * a) * (t * t)


@jax.jit
def kernel(pred_logits, gt_label, gt_score):
    N, C = pred_logits.shape
    BN = 20480
    grid = (pl.cdiv(N, BN),)
    pt = pred_logits.T
    gt = gt_score.T
    lab = gt_label.astype(jnp.int32)
    out_t = pl.pallas_call(
        _qfl_block_t,
        grid=grid,
        in_specs=[
            pl.BlockSpec((C, BN), lambda i: (0, i)),
            pl.BlockSpec((C, BN), lambda i: (0, i)),
            pl.BlockSpec((BN,), lambda i: (i,)),
        ],
        out_specs=pl.BlockSpec((C, BN), lambda i: (0, i)),
        out_shape=jax.ShapeDtypeStruct((C, N), jnp.float32),
    )(pt, gt, lab)
    return out_t.T
